# baseline (speedup 1.0000x reference)
import jax
import jax.numpy as jnp
from jax import lax
from jax.experimental import pallas as pl
from jax.experimental.pallas import tpu as pltpu
from jax.experimental.pallas import tpu_sc as plsc

N = 10000
E = 320000
D_IN = 128
D_EDGE = 16
H = 128
L = 5
ML = 4
G = 64

NC = 2
NS = 16
NW = NC * NS
EPW = E // NW
C = 80
NCHUNK = EPW // C
N_PAD = 10240
RPT = N_PAD // NS
ZR = 128


NB = 3
ND = 6


def _sc_edge_body(h_hbm, e_hbm, src_hbm, dst_hbm, agg_hbm, *refs):
    srcv = refs[0:NB]
    dstv = refs[NB:NB + ND]
    ebuf = refs[NB + ND:2 * NB + ND]
    hbuf = refs[2 * NB + ND:3 * NB + ND]
    acc = refs[3 * NB + ND]
    psem = refs[3 * NB + ND + 1:3 * NB + ND + 1 + NB]
    gsem = refs[3 * NB + ND + 1 + NB:3 * NB + ND + 1 + 2 * NB]
    ssem = refs[3 * NB + ND + 1 + 2 * NB:3 * NB + ND + 1 + 3 * NB]
    cid = lax.axis_index("c")
    sid = lax.axis_index("s")
    wid = sid * NC + cid

    def _zrow(r, carry):
        for c8 in range(H // 16):
            hbuf[0][r, pl.ds(c8 * 16, 16)] = jnp.zeros((16,), jnp.float32)
        return carry
    lax.fori_loop(0, C, _zrow, 0)
    for j in range(RPT // C):
        pltpu.sync_copy(hbuf[0], acc.at[pl.ds(sid * RPT + j * C, C)])
    plsc.subcore_barrier()

    def _start_pre(k, b, d):
        base = wid * EPW + k * C
        ebase = wid * (EPW // 2) + k * (C // 2)
        pltpu.async_copy(src_hbm.at[pl.ds(base, C)], srcv[b], psem[b])
        pltpu.async_copy(dst_hbm.at[pl.ds(base, C)], dstv[d], psem[b])
        pltpu.async_copy(e_hbm.at[pl.ds(ebase, C // 2)], ebuf[b], psem[b])

    def _wait_pre(k, b, d):
        base = wid * EPW + k * C
        ebase = wid * (EPW // 2) + k * (C // 2)
        pltpu.make_async_copy(src_hbm.at[pl.ds(base, C)], srcv[b], psem[b]).wait()
        pltpu.make_async_copy(dst_hbm.at[pl.ds(base, C)], dstv[d], psem[b]).wait()
        pltpu.make_async_copy(e_hbm.at[pl.ds(ebase, C // 2)], ebuf[b],
                              psem[b]).wait()

    def _gather_start(b):
        pltpu.async_copy(h_hbm.at[srcv[b]], hbuf[b], gsem[b])

    def _gather_wait(b):
        pltpu.make_async_copy(h_hbm.at[srcv[b]], hbuf[b], gsem[b]).wait()

    def _cvt_add_relu(b):
        @plsc.parallel_loop(0, C // 2, step=1, unroll=2)
        def _rowpair(rp):
            r = 2 * rp
            for m in range(H // 16):
                s = pl.ds(16 * m, 16)
                v = ebuf[b][rp, s]
                lo = lax.bitcast_convert_type(v << 16, jnp.float32)
                hi = lax.bitcast_convert_type(v & jnp.uint32(0xFFFF0000),
                                              jnp.float32)
                hbuf[b][r, s] = jnp.maximum(hbuf[b][r, s] + lo, 0.0)
                hbuf[b][r + 1, s] = jnp.maximum(hbuf[b][r + 1, s] + hi, 0.0)

    def _scatter_start(b, d):
        pltpu.async_copy(hbuf[b], acc.at[dstv[d]], ssem[b], add=True)

    def _scatter_wait(b, d):
        pltpu.make_async_copy(hbuf[b], acc.at[dstv[d]], ssem[b]).wait()

    def _do_chunk(k, b, d):
        _gather_wait(b)
        _wait_pre(k + 2, (b + 2) % NB, (d + 2) % ND)
        _cvt_add_relu(b)
        _scatter_wait((b + 2) % NB, (d + 5) % ND)
        _gather_start((b + 2) % NB)
        _scatter_start(b, d)
        _start_pre(k + 3, b, (d + 3) % ND)

    for k0 in (0, 1, 2):
        _start_pre(k0, k0, k0)
    for k0 in (0, 1):
        _wait_pre(k0, k0, k0)
        _gather_start(k0)
    _gather_wait(0)
    _wait_pre(2, 2, 2)
    _gather_start(2)
    _cvt_add_relu(0)
    _scatter_start(0, 0)
    _start_pre(3, 0, 3)
    _gather_wait(1)
    _wait_pre(3, 0, 3)
    _scatter_wait(0, 0)
    _gather_start(0)
    _cvt_add_relu(1)
    _scatter_start(1, 1)
    _start_pre(4, 1, 4)

    def _six(i, carry):
        k = ND * i + 2
        for j in range(ND):
            _do_chunk(k + j, (2 + j) % NB, (2 + j) % ND)
        return carry
    lax.fori_loop(0, (NCHUNK - 5) // ND, _six, 0)

    _gather_wait(2)
    _wait_pre(NCHUNK - 1, 1, 4)
    _scatter_wait(1, 1)
    _gather_start(1)
    _cvt_add_relu(2)
    _scatter_start(2, 2)
    _gather_wait(0)
    _scatter_wait(2, 2)
    _cvt_add_relu(0)
    _scatter_start(0, 3)
    _gather_wait(1)
    _scatter_wait(0, 3)
    _cvt_add_relu(1)
    _scatter_start(1, 4)
    _scatter_wait(1, 4)

    plsc.subcore_barrier()
    pltpu.sync_copy(acc.at[pl.ds(sid * RPT, RPT)],
                    agg_hbm.at[cid, pl.ds(sid * RPT, RPT)])


_sc_edge = pl.kernel(
    _sc_edge_body,
    out_type=jax.ShapeDtypeStruct((NC, N_PAD, H), jnp.float32),
    mesh=plsc.VectorSubcoreMesh(core_axis_name="c", subcore_axis_name="s"),
    scratch_types=(
        [pltpu.VMEM((C,), jnp.int32)] * NB
        + [pltpu.VMEM((C,), jnp.int32)] * ND
        + [pltpu.VMEM((C // 2, H), jnp.uint32)] * NB
        + [pltpu.VMEM((C, H), jnp.float32)] * NB
        + [
            pltpu.VMEM_SHARED((N_PAD, H), jnp.float32),
        ]
        + [pltpu.SemaphoreType.DMA] * (3 * NB)
    ),
    name="sc_gine_edge",
)


def _node_proj_body(x_ref, w_ref, b_ref, o_ref):
    o_ref[...] = (jnp.dot(x_ref[...], w_ref[...],
                          preferred_element_type=jnp.float32) + b_ref[...])


def _edge_proj_body(a_ref, w_ref, b_ref, o_ref):
    out = (jnp.dot(a_ref[...], w_ref[...],
                   preferred_element_type=jnp.float32)
           + b_ref[...]).astype(jnp.bfloat16)
    u = lax.bitcast_convert_type(out, jnp.uint16).astype(jnp.uint32)
    u = u.reshape(BE // 2, 2, H)
    o_ref[...] = u[:, 0, :] | (u[:, 1, :] << 16)


def _layer_body(h_ref, agg_ref, w_ref, b_ref, g_ref, bt_ref, eps_ref, o_ref):
    h = h_ref[...]
    out = (1.0 + eps_ref[0]) * h + agg_ref[0] + agg_ref[1]
    for j in range(ML):
        out = jnp.dot(out, w_ref[j], preferred_element_type=jnp.float32) + b_ref[j]
        if j < ML - 1:
            out = jnp.maximum(out, 0.0)
    mu = jnp.mean(out, axis=0, keepdims=True)
    var = jnp.mean((out - mu) ** 2, axis=0, keepdims=True)
    out = g_ref[...] * (out - mu) / jnp.sqrt(var + 1e-5) + bt_ref[...]
    o_ref[...] = jnp.maximum(out, 0.0) + h


def _last_layer_pool_body(h_ref, agg_ref, w_ref, b_ref, g_ref, bt_ref,
                          eps_ref, batch_ref, o_ref):
    h = h_ref[...]
    out = (1.0 + eps_ref[0]) * h + agg_ref[0] + agg_ref[1]
    for j in range(ML):
        out = jnp.dot(out, w_ref[j], preferred_element_type=jnp.float32) + b_ref[j]
        if j < ML - 1:
            out = jnp.maximum(out, 0.0)
    mu = jnp.mean(out, axis=0, keepdims=True)
    var = jnp.mean((out - mu) ** 2, axis=0, keepdims=True)
    out = g_ref[...] * (out - mu) / jnp.sqrt(var + 1e-5) + bt_ref[...]
    out = jnp.maximum(out, 0.0) + h
    onehot = (batch_ref[...] ==
              lax.broadcasted_iota(jnp.int32, (1, G), 1)).astype(jnp.float32)
    sums = lax.dot_general(onehot, out, (((0,), (0,)), ((), ())),
                           preferred_element_type=jnp.float32)
    counts = lax.dot_general(onehot, jnp.ones((N, 1), jnp.float32),
                             (((0,), (0,)), ((), ())),
                             preferred_element_type=jnp.float32)
    o_ref[...] = sums / jnp.maximum(counts, 1.0)


BE = 8000


def kernel(x, edge_index, batch, edge_attr, Wn, bn_lin, We, be, eps,
           mlp_W, mlp_b, bn_gamma, bn_beta):
    src = edge_index[0].astype(jnp.int32)
    dst = edge_index[1].astype(jnp.int32)

    h = pl.pallas_call(
        _node_proj_body,
        out_shape=jax.ShapeDtypeStruct((N, H), jnp.float32),
    )(x, Wn, bn_lin.reshape(1, H))

    e = pl.pallas_call(
        _edge_proj_body,
        grid=(E // BE,),
        in_specs=[
            pl.BlockSpec((BE, D_EDGE), lambda i: (i, 0)),
            pl.BlockSpec((D_EDGE, H), lambda i: (0, 0)),
            pl.BlockSpec((1, H), lambda i: (0, 0)),
        ],
        out_specs=pl.BlockSpec((BE // 2, H), lambda i: (i, 0)),
        out_shape=jax.ShapeDtypeStruct((E // 2, H), jnp.uint32),
    )(edge_attr, We, be.reshape(1, H))

    layer_call = pl.pallas_call(
        _layer_body,
        grid=(1,),
        in_specs=[
            pl.BlockSpec((N, H), lambda i: (0, 0)),
            pl.BlockSpec((NC, N, H), lambda i: (0, 0, 0)),
            pl.BlockSpec((ML, H, H), lambda i: (0, 0, 0)),
            pl.BlockSpec((ML, 1, H), lambda i: (0, 0, 0)),
            pl.BlockSpec((1, H), lambda i: (0, 0)),
            pl.BlockSpec((1, H), lambda i: (0, 0)),
            pl.BlockSpec(memory_space=pltpu.SMEM),
        ],
        out_specs=pl.BlockSpec((N, H), lambda i: (0, 0)),
        out_shape=jax.ShapeDtypeStruct((N, H), jnp.float32),
    )

    for i in range(L - 1):
        agg = _sc_edge(h, e, src, dst)
        h = layer_call(h, agg, mlp_W[i], mlp_b[i].reshape(ML, 1, H),
                       bn_gamma[i].reshape(1, H), bn_beta[i].reshape(1, H),
                       eps[i].reshape(1))

    agg = _sc_edge(h, e, src, dst)
    i = L - 1
    g = pl.pallas_call(
        _last_layer_pool_body,
        grid=(1,),
        in_specs=[
            pl.BlockSpec((N, H), lambda i: (0, 0)),
            pl.BlockSpec((NC, N, H), lambda i: (0, 0, 0)),
            pl.BlockSpec((ML, H, H), lambda i: (0, 0, 0)),
            pl.BlockSpec((ML, 1, H), lambda i: (0, 0, 0)),
            pl.BlockSpec((1, H), lambda i: (0, 0)),
            pl.BlockSpec((1, H), lambda i: (0, 0)),
            pl.BlockSpec(memory_space=pltpu.SMEM),
            pl.BlockSpec((N, 1), lambda i: (0, 0)),
        ],
        out_specs=pl.BlockSpec((G, H), lambda i: (0, 0)),
        out_shape=jax.ShapeDtypeStruct((G, H), jnp.float32),
    )(h, agg, mlp_W[i], mlp_b[i].reshape(ML, 1, H),
      bn_gamma[i].reshape(1, H), bn_beta[i].reshape(1, H),
      eps[i].reshape(1), batch.astype(jnp.int32).reshape(N, 1))
    return g


# speedup vs baseline: 1.0924x; 1.0924x over previous
import jax
import jax.numpy as jnp
from jax import lax
from jax.experimental import pallas as pl
from jax.experimental.pallas import tpu as pltpu
from jax.experimental.pallas import tpu_sc as plsc

N = 10000
E = 320000
D_IN = 128
D_EDGE = 16
H = 128
L = 5
ML = 4
G = 64

NC = 2
NS = 16
NW = NC * NS
EPW = E // NW
C = 80
NCHUNK = EPW // C
N_PAD = 10240
RPT = N_PAD // NS
ZR = 128


NB = 3
ND = 6


def _sc_edge_body(h_hbm, e_hbm, src_hbm, dst_hbm, agg_hbm, *refs):
    srcv = refs[0:NB]
    dstv = refs[NB:NB + ND]
    ebuf = refs[NB + ND:2 * NB + ND]
    hbuf = refs[2 * NB + ND:3 * NB + ND]
    acc = refs[3 * NB + ND]
    psem = refs[3 * NB + ND + 1:3 * NB + ND + 1 + NB]
    gsem = refs[3 * NB + ND + 1 + NB:3 * NB + ND + 1 + 2 * NB]
    ssem = refs[3 * NB + ND + 1 + 2 * NB:3 * NB + ND + 1 + 3 * NB]
    cid = lax.axis_index("c")
    sid = lax.axis_index("s")
    wid = sid * NC + cid

    def _zrow(r, carry):
        for c8 in range(H // 16):
            hbuf[0][r, pl.ds(c8 * 16, 16)] = jnp.zeros((16,), jnp.float32)
        return carry
    lax.fori_loop(0, C, _zrow, 0)
    for j in range(RPT // C):
        pltpu.sync_copy(hbuf[0], acc.at[pl.ds(sid * RPT + j * C, C)])
    plsc.subcore_barrier()

    def _start_pre(k, b, d):
        base = wid * EPW + k * C
        ebase = wid * (EPW // 2) + k * (C // 2)
        pltpu.async_copy(src_hbm.at[pl.ds(base, C)], srcv[b], psem[b])
        pltpu.async_copy(dst_hbm.at[pl.ds(base, C)], dstv[d], psem[b])
        pltpu.async_copy(e_hbm.at[pl.ds(ebase, C // 2)], ebuf[b], psem[b])

    def _wait_pre(k, b, d):
        base = wid * EPW + k * C
        ebase = wid * (EPW // 2) + k * (C // 2)
        pltpu.make_async_copy(src_hbm.at[pl.ds(base, C)], srcv[b], psem[b]).wait()
        pltpu.make_async_copy(dst_hbm.at[pl.ds(base, C)], dstv[d], psem[b]).wait()
        pltpu.make_async_copy(e_hbm.at[pl.ds(ebase, C // 2)], ebuf[b],
                              psem[b]).wait()

    def _gather_start(b):
        pltpu.async_copy(h_hbm.at[srcv[b]], hbuf[b], gsem[b])

    def _gather_wait(b):
        pltpu.make_async_copy(h_hbm.at[srcv[b]], hbuf[b], gsem[b]).wait()

    def _cvt_add_relu(b):
        @plsc.parallel_loop(0, C // 2, step=1, unroll=2)
        def _rowpair(rp):
            r = 2 * rp
            for m in range(H // 16):
                s = pl.ds(16 * m, 16)
                v = ebuf[b][rp, s]
                lo = lax.bitcast_convert_type(v << 16, jnp.float32)
                hi = lax.bitcast_convert_type(v & jnp.uint32(0xFFFF0000),
                                              jnp.float32)
                hbuf[b][r, s] = jnp.maximum(hbuf[b][r, s] + lo, 0.0)
                hbuf[b][r + 1, s] = jnp.maximum(hbuf[b][r + 1, s] + hi, 0.0)

    def _scatter_start(b, d):
        pltpu.async_copy(hbuf[b], acc.at[dstv[d]], ssem[b], add=True)

    def _scatter_wait(b, d):
        pltpu.make_async_copy(hbuf[b], acc.at[dstv[d]], ssem[b]).wait()

    def _do_chunk(k, b, d):
        _gather_wait(b)
        _wait_pre(k + 2, (b + 2) % NB, (d + 2) % ND)
        _scatter_wait((b + 2) % NB, (d + 5) % ND)
        _gather_start((b + 2) % NB)
        _cvt_add_relu(b)
        _scatter_start(b, d)
        _start_pre(k + 3, b, (d + 3) % ND)

    for k0 in (0, 1, 2):
        _start_pre(k0, k0, k0)
    for k0 in (0, 1):
        _wait_pre(k0, k0, k0)
        _gather_start(k0)
    _gather_wait(0)
    _wait_pre(2, 2, 2)
    _gather_start(2)
    _cvt_add_relu(0)
    _scatter_start(0, 0)
    _start_pre(3, 0, 3)
    _gather_wait(1)
    _wait_pre(3, 0, 3)
    _scatter_wait(0, 0)
    _gather_start(0)
    _cvt_add_relu(1)
    _scatter_start(1, 1)
    _start_pre(4, 1, 4)

    def _six(i, carry):
        k = ND * i + 2
        for j in range(ND):
            _do_chunk(k + j, (2 + j) % NB, (2 + j) % ND)
        return carry
    lax.fori_loop(0, (NCHUNK - 5) // ND, _six, 0)

    _gather_wait(2)
    _wait_pre(NCHUNK - 1, 1, 4)
    _scatter_wait(1, 1)
    _gather_start(1)
    _cvt_add_relu(2)
    _scatter_start(2, 2)
    _gather_wait(0)
    _scatter_wait(2, 2)
    _cvt_add_relu(0)
    _scatter_start(0, 3)
    _gather_wait(1)
    _scatter_wait(0, 3)
    _cvt_add_relu(1)
    _scatter_start(1, 4)
    _scatter_wait(1, 4)

    plsc.subcore_barrier()
    pltpu.sync_copy(acc.at[pl.ds(sid * RPT, RPT)],
                    agg_hbm.at[cid, pl.ds(sid * RPT, RPT)])


_sc_edge = pl.kernel(
    _sc_edge_body,
    out_type=jax.ShapeDtypeStruct((NC, N_PAD, H), jnp.float32),
    mesh=plsc.VectorSubcoreMesh(core_axis_name="c", subcore_axis_name="s"),
    scratch_types=(
        [pltpu.VMEM((C,), jnp.int32)] * NB
        + [pltpu.VMEM((C,), jnp.int32)] * ND
        + [pltpu.VMEM((C // 2, H), jnp.uint32)] * NB
        + [pltpu.VMEM((C, H), jnp.float32)] * NB
        + [
            pltpu.VMEM_SHARED((N_PAD, H), jnp.float32),
        ]
        + [pltpu.SemaphoreType.DMA] * (3 * NB)
    ),
    name="sc_gine_edge",
)


def _node_proj_body(x_ref, w_ref, b_ref, o_ref):
    o_ref[...] = (jnp.dot(x_ref[...], w_ref[...],
                          preferred_element_type=jnp.float32) + b_ref[...])


def _edge_proj_body(a_ref, w_ref, b_ref, o_ref):
    out = (jnp.dot(a_ref[...], w_ref[...],
                   preferred_element_type=jnp.float32)
           + b_ref[...]).astype(jnp.bfloat16)
    u = lax.bitcast_convert_type(out, jnp.uint16).astype(jnp.uint32)
    u = u.reshape(BE // 2, 2, H)
    o_ref[...] = u[:, 0, :] | (u[:, 1, :] << 16)


def _layer_body(h_ref, agg_ref, w_ref, b_ref, g_ref, bt_ref, eps_ref, o_ref):
    h = h_ref[...]
    out = (1.0 + eps_ref[0]) * h + agg_ref[0] + agg_ref[1]
    for j in range(ML):
        out = jnp.dot(out, w_ref[j], preferred_element_type=jnp.float32) + b_ref[j]
        if j < ML - 1:
            out = jnp.maximum(out, 0.0)
    mu = jnp.mean(out, axis=0, keepdims=True)
    var = jnp.mean((out - mu) ** 2, axis=0, keepdims=True)
    out = g_ref[...] * (out - mu) / jnp.sqrt(var + 1e-5) + bt_ref[...]
    o_ref[...] = jnp.maximum(out, 0.0) + h


def _last_layer_pool_body(h_ref, agg_ref, w_ref, b_ref, g_ref, bt_ref,
                          eps_ref, batch_ref, o_ref):
    h = h_ref[...]
    out = (1.0 + eps_ref[0]) * h + agg_ref[0] + agg_ref[1]
    for j in range(ML):
        out = jnp.dot(out, w_ref[j], preferred_element_type=jnp.float32) + b_ref[j]
        if j < ML - 1:
            out = jnp.maximum(out, 0.0)
    mu = jnp.mean(out, axis=0, keepdims=True)
    var = jnp.mean((out - mu) ** 2, axis=0, keepdims=True)
    out = g_ref[...] * (out - mu) / jnp.sqrt(var + 1e-5) + bt_ref[...]
    out = jnp.maximum(out, 0.0) + h
    onehot = (batch_ref[...] ==
              lax.broadcasted_iota(jnp.int32, (1, G), 1)).astype(jnp.float32)
    sums = lax.dot_general(onehot, out, (((0,), (0,)), ((), ())),
                           preferred_element_type=jnp.float32)
    counts = lax.dot_general(onehot, jnp.ones((N, 1), jnp.float32),
                             (((0,), (0,)), ((), ())),
                             preferred_element_type=jnp.float32)
    o_ref[...] = sums / jnp.maximum(counts, 1.0)


BE = 8000


def kernel(x, edge_index, batch, edge_attr, Wn, bn_lin, We, be, eps,
           mlp_W, mlp_b, bn_gamma, bn_beta):
    src = edge_index[0].astype(jnp.int32)
    dst = edge_index[1].astype(jnp.int32)

    h = pl.pallas_call(
        _node_proj_body,
        out_shape=jax.ShapeDtypeStruct((N, H), jnp.float32),
    )(x, Wn, bn_lin.reshape(1, H))

    e = pl.pallas_call(
        _edge_proj_body,
        grid=(E // BE,),
        in_specs=[
            pl.BlockSpec((BE, D_EDGE), lambda i: (i, 0)),
            pl.BlockSpec((D_EDGE, H), lambda i: (0, 0)),
            pl.BlockSpec((1, H), lambda i: (0, 0)),
        ],
        out_specs=pl.BlockSpec((BE // 2, H), lambda i: (i, 0)),
        out_shape=jax.ShapeDtypeStruct((E // 2, H), jnp.uint32),
    )(edge_attr, We, be.reshape(1, H))

    layer_call = pl.pallas_call(
        _layer_body,
        grid=(1,),
        in_specs=[
            pl.BlockSpec((N, H), lambda i: (0, 0)),
            pl.BlockSpec((NC, N, H), lambda i: (0, 0, 0)),
            pl.BlockSpec((ML, H, H), lambda i: (0, 0, 0)),
            pl.BlockSpec((ML, 1, H), lambda i: (0, 0, 0)),
            pl.BlockSpec((1, H), lambda i: (0, 0)),
            pl.BlockSpec((1, H), lambda i: (0, 0)),
            pl.BlockSpec(memory_space=pltpu.SMEM),
        ],
        out_specs=pl.BlockSpec((N, H), lambda i: (0, 0)),
        out_shape=jax.ShapeDtypeStruct((N, H), jnp.float32),
    )

    for i in range(L - 1):
        agg = _sc_edge(h, e, src, dst)
        h = layer_call(h, agg, mlp_W[i], mlp_b[i].reshape(ML, 1, H),
                       bn_gamma[i].reshape(1, H), bn_beta[i].reshape(1, H),
                       eps[i].reshape(1))

    agg = _sc_edge(h, e, src, dst)
    i = L - 1
    g = pl.pallas_call(
        _last_layer_pool_body,
        grid=(1,),
        in_specs=[
            pl.BlockSpec((N, H), lambda i: (0, 0)),
            pl.BlockSpec((NC, N, H), lambda i: (0, 0, 0)),
            pl.BlockSpec((ML, H, H), lambda i: (0, 0, 0)),
            pl.BlockSpec((ML, 1, H), lambda i: (0, 0, 0)),
            pl.BlockSpec((1, H), lambda i: (0, 0)),
            pl.BlockSpec((1, H), lambda i: (0, 0)),
            pl.BlockSpec(memory_space=pltpu.SMEM),
            pl.BlockSpec((N, 1), lambda i: (0, 0)),
        ],
        out_specs=pl.BlockSpec((G, H), lambda i: (0, 0)),
        out_shape=jax.ShapeDtypeStruct((G, H), jnp.float32),
    )(h, agg, mlp_W[i], mlp_b[i].reshape(ML, 1, H),
      bn_gamma[i].reshape(1, H), bn_beta[i].reshape(1, H),
      eps[i].reshape(1), batch.astype(jnp.int32).reshape(N, 1))
    return g
